# baseline (speedup 1.0000x reference)
import functools

import jax
import jax.numpy as jnp
import numpy as np
from jax import lax
from jax.experimental import pallas as pl
from jax.experimental.pallas import tpu as pltpu
from jax.experimental.pallas import tpu_sc as plsc

N = 10000
H = 32
IN_FDIM = 128
OUT_FDIM = 128
MID = 32
NUM_KP = 15
K16 = 16
TW = 48
KP_EXTENT = 1.0

NC, NS = 2, 16
NW = NC * NS
CH = 2000
NCHUNK = N // CH

BLK1 = 2000
BLK2 = 400

_HI = lax.Precision.HIGHEST


def _leaky(x):
    return jnp.where(x > 0, x, 0.1 * x)


def _kp_constants():
    rng = np.random.RandomState(42)
    kp = rng.uniform(-1.0, 1.0, size=(NUM_KP, 3)).astype(np.float32)
    kp = kp / np.maximum(np.linalg.norm(kp, axis=1, keepdims=True), 1e-6)
    radii = rng.uniform(0.0, 1.0, size=(NUM_KP, 1)).astype(np.float32) ** (1.0 / 3.0)
    kp = kp * radii * 0.66 * KP_EXTENT
    kp[0, :] = 0.0
    kpm = np.zeros((8, K16), dtype=np.float32)
    kpm[0:3, :NUM_KP] = kp.T
    kpm[3, :NUM_KP] = -0.5 * np.sum(kp * kp, axis=1)
    return kpm


_KPM = _kp_constants()

_E = np.zeros((K16, NUM_KP * MID), dtype=np.float32)
for _k in range(NUM_KP):
    _E[_k, _k * MID:(_k + 1) * MID] = 1.0
_R = np.zeros((MID, NUM_KP * MID), dtype=np.float32)
for _k in range(NUM_KP):
    _R[np.arange(MID), _k * MID + np.arange(MID)] = 1.0


def _tc1_body(feat, w1, spa, t_out):
    x1 = _leaky(jnp.dot(feat[...], w1[...], precision=_HI))
    t_out[...] = jnp.concatenate([x1, spa[...]], axis=1)


def _build_table(features, W1, sp_aug):
    return pl.pallas_call(
        _tc1_body,
        grid=(N // BLK1,),
        in_specs=[
            pl.BlockSpec((BLK1, IN_FDIM), lambda i: (i, 0)),
            pl.BlockSpec((IN_FDIM, MID), lambda i: (0, 0)),
            pl.BlockSpec((BLK1, 16), lambda i: (i, 0)),
        ],
        out_specs=pl.BlockSpec((BLK1, TW), lambda i: (i, 0)),
        out_shape=jax.ShapeDtypeStruct((N, TW), jnp.float32),
    )(features, W1, sp_aug)


def _sc_body(t_hbm, idxt_hbm, g_hbm, idx_v, rows_v, sem):
    wid = lax.axis_index("s") * NC + lax.axis_index("c")
    for ci in range(NCHUNK):
        base = ci * CH
        pltpu.sync_copy(idxt_hbm.at[pl.ds(wid * N + base, CH)], idx_v)
        pltpu.async_copy(t_hbm.at[idx_v], rows_v, sem).wait()
        pltpu.sync_copy(rows_v, g_hbm.at[wid, pl.ds(base, CH), :])


@functools.cache
def _sc_gather_fn():
    return functools.partial(
        pl.kernel,
        out_type=jax.ShapeDtypeStruct((H, N, TW), jnp.float32),
        mesh=plsc.VectorSubcoreMesh(core_axis_name="c", subcore_axis_name="s",
                                    num_cores=NC, num_subcores=NS),
        scratch_types=[
            pltpu.VMEM((CH,), jnp.int32),
            pltpu.VMEM((CH, TW), jnp.float32),
            pltpu.SemaphoreType.DMA,
        ],
        compiler_params=pltpu.CompilerParams(use_tc_tiling_on_sc=False),
    )(_sc_body)


def _tc2_body(g, qp8, feat, w2r, w3, ws, kpm, e, r, out):
    qb = qp8[...]
    acc = jnp.zeros((BLK2, NUM_KP * MID), jnp.float32)
    for h in range(H):
        gh = g[h]
        d8 = gh[:, 32:40] - qb
        nd2 = jnp.sum(d8 * d8, axis=1, keepdims=True) - 1.0
        sq = nd2 - 2.0 * jnp.dot(d8, kpm[...], precision=_HI)
        w = jnp.maximum(1.0 - jnp.sqrt(jnp.maximum(sq, 0.0) + 1e-12), 0.0)
        nx = gh[:, 0:32]
        acc = acc + (jnp.dot(w, e[...], precision=_HI)
                     * jnp.dot(nx, r[...], precision=_HI))
    x2 = _leaky(jnp.dot(acc, w2r[...], precision=_HI))
    x3 = _leaky(jnp.dot(x2, w3[...], precision=_HI))
    sc = _leaky(jnp.dot(feat[...], ws[...], precision=_HI))
    out[...] = _leaky(sc + x3)


def _aggregate(G, qp8, features, W2r, W3, Ws):
    return pl.pallas_call(
        _tc2_body,
        grid=(N // BLK2,),
        in_specs=[
            pl.BlockSpec((H, BLK2, TW), lambda i: (0, i, 0)),
            pl.BlockSpec((BLK2, 8), lambda i: (i, 0)),
            pl.BlockSpec((BLK2, IN_FDIM), lambda i: (i, 0)),
            pl.BlockSpec((NUM_KP * MID, MID), lambda i: (0, 0)),
            pl.BlockSpec((MID, OUT_FDIM), lambda i: (0, 0)),
            pl.BlockSpec((IN_FDIM, OUT_FDIM), lambda i: (0, 0)),
            pl.BlockSpec((8, K16), lambda i: (0, 0)),
            pl.BlockSpec((K16, NUM_KP * MID), lambda i: (0, 0)),
            pl.BlockSpec((MID, NUM_KP * MID), lambda i: (0, 0)),
        ],
        out_specs=pl.BlockSpec((BLK2, OUT_FDIM), lambda i: (i, 0)),
        out_shape=jax.ShapeDtypeStruct((N, OUT_FDIM), jnp.float32),
    )(G, qp8, features, W2r, W3, Ws, jnp.asarray(_KPM), jnp.asarray(_E),
      jnp.asarray(_R))


def kernel(query_points, support_points, neighbors_indices, features, W1, W2, W3, Ws):
    sp_aug = jnp.concatenate(
        [support_points,
         jnp.ones((N, 1), jnp.float32),
         jnp.zeros((N, 12), jnp.float32)], axis=1)
    qp8 = jnp.concatenate([query_points, jnp.zeros((N, 5), jnp.float32)], axis=1)
    nidx_flat = jnp.transpose(neighbors_indices).reshape(-1)

    T = _build_table(features, W1, sp_aug)
    G = _sc_gather_fn()(T, nidx_flat)
    W2r = W2.reshape(NUM_KP * MID, MID)
    return _aggregate(G, qp8, features, W2r, W3, Ws)
